# Initial kernel scaffold; baseline (speedup 1.0000x reference)
import jax
import jax.numpy as jnp
from jax.experimental import pallas as pl


def kernel(edge_attr, edge_index, W0, b0, W1, b1, W2, b2):
    raise NotImplementedError("write your pallas kernel here")


# speedup vs baseline: 3.8820x; 3.8820x over previous
import jax
import jax.numpy as jnp
from jax import lax
from jax.experimental import pallas as pl
from jax.experimental.pallas import tpu as pltpu
from jax.experimental.pallas import tpu_sc as plsc

_SC_PARAMS = pltpu.CompilerParams(needs_layout_passes=False)

E = 320000
N = 10000
F = 128
NP = 10240

NC = 2
NS = 16
L = 16

C1 = 40
NCH = (E // NS) // C1
NSLAB = 4
QTR = NCH // NSLAB
RPS = NP // NS

C3 = 80
NW = NC * NS
EPW = E // NW
ICH = (2 * EPW) // C3


def _seg_sum_kernel(ea_hbm, idx_hbm, acc_hbm, hist_hbm,
                    acc, idx_v, data_v, hist_v, sem0, sem1):
    c = lax.axis_index("core")
    s = lax.axis_index("subcore")
    cw = c * NS + s
    lane = lax.iota(jnp.int32, L)
    ones = jnp.ones((L,), jnp.float32)

    @pl.loop(0, C1)
    def _(i):
        @pl.loop(0, F, step=L)
        def _(l):
            data_v[0, i, pl.ds(l, L)] = jnp.zeros((L,), jnp.float32)

    @pl.loop(0, NP, step=L)
    def _(i):
        hist_v[0, pl.ds(i, L)] = jnp.zeros((L,), jnp.float32)

    @pl.loop(0, RPS // C1)
    def _(k):
        pltpu.sync_copy(data_v.at[0],
                        acc.at[pl.ds(s * RPS + k * C1, C1)])
    plsc.subcore_barrier()

    def start(g, buf, sem):
        pltpu.async_copy(ea_hbm.at[pl.ds((s * NCH + g) * C1, C1)],
                         data_v.at[buf], sem)

    def wait(buf, sem):
        pltpu.make_async_copy(ea_hbm.at[pl.ds(0, C1)],
                              data_v.at[buf], sem).wait()

    def scatter(j, buf):
        pltpu.sync_copy(data_v.at[buf], acc.at[idx_v.at[j]], add=True)

    for h in range(NSLAB):
        pltpu.sync_copy(idx_hbm.at[cw * NSLAB + h], idx_v)
        base = h * QTR
        start(base + 0, 0, sem0)
        start(base + 1, 1, sem1)

        @pl.loop(0, (QTR + 1) // 2)
        def _(k):
            wait(0, sem0)
            scatter(2 * k, 0)

            @pl.when(2 * k + 2 < QTR)
            def _():
                start(base + 2 * k + 2, 0, sem0)

            @pl.when(2 * k + 1 < QTR)
            def _():
                wait(1, sem1)
                scatter(2 * k + 1, 1)

                @pl.when(2 * k + 3 < QTR)
                def _():
                    start(base + 2 * k + 3, 1, sem1)

        @pl.loop(0, (QTR * C1 + L - 1) // L)
        def _(i):
            p = i * L + lane
            msk = p < QTR * C1
            vidx = plsc.load_gather(idx_v, [p // C1, p % C1], mask=msk)
            plsc.addupdate_scatter(hist_v, [jnp.zeros((L,), jnp.int32), vidx],
                                   ones, mask=msk)

    plsc.subcore_barrier()

    @pl.loop(0, RPS // C1)
    def _(k):
        pltpu.sync_copy(acc.at[pl.ds(s * RPS + k * C1, C1)], data_v.at[0])
        pltpu.sync_copy(data_v.at[0], acc_hbm.at[cw * (RPS // C1) + k])
    pltpu.sync_copy(hist_v, hist_hbm.at[cw])


def _seg_sums(edge_attr, idx_chunks):
    mesh = plsc.VectorSubcoreMesh(core_axis_name="core",
                                  subcore_axis_name="subcore")
    nslab = NC * NS * (RPS // C1)
    f = pl.kernel(
        _seg_sum_kernel,
        out_type=(jax.ShapeDtypeStruct((nslab, C1, F), jnp.float32),
                  jax.ShapeDtypeStruct((NC * NS, 1, NP), jnp.float32)),
        mesh=mesh,
        scratch_types=[
            pltpu.VMEM_SHARED((NP, F), jnp.float32),
            pltpu.VMEM((QTR, C1), jnp.int32),
            pltpu.VMEM((2, C1, F), jnp.float32),
            pltpu.VMEM((1, NP), jnp.float32),
            pltpu.SemaphoreType.DMA,
            pltpu.SemaphoreType.DMA,
        ],
        compiler_params=_SC_PARAMS,
    )
    accs, hist = f(edge_attr, idx_chunks)
    return accs.reshape(2 * NP, F), hist.reshape(NC * NS, NP)


BLK = 640


def _dense_kernel(sum_src, sum_dst, hist_src, hist_dst,
                  w0t, b0, w1t, b1, w2t, b2, h1_out, tt_out):
    cnt_src = jnp.sum(hist_src[...], axis=0).reshape(BLK, 1)
    cnt_dst = jnp.sum(hist_dst[...], axis=0).reshape(BLK, 1)
    mean_src = sum_src[...] / jnp.maximum(cnt_src, 1.0)
    mean_dst = sum_dst[...] / jnp.maximum(cnt_dst, 1.0)
    w0 = w0t[...]
    z0 = (jnp.dot(mean_dst, w0[:F], preferred_element_type=jnp.float32)
          + jnp.dot(mean_src, w0[F:], preferred_element_type=jnp.float32)
          + b0[...])
    h0 = 1.0 / (1.0 + jnp.exp(-z0))
    w1 = w1t[...]
    z1 = (jnp.dot(h0, w1[:F], preferred_element_type=jnp.float32)
          + jnp.dot(mean_src, w1[F:], preferred_element_type=jnp.float32)
          + b1[...])
    h1 = 1.0 / (1.0 + jnp.exp(-z1))
    h1_out[...] = h1
    w2 = w2t[...]
    sa = jnp.dot(h1, w2[:F], preferred_element_type=jnp.float32) + b2[...]
    sb = jnp.dot(h1, w2[F:], preferred_element_type=jnp.float32)
    tt_out[...] = jnp.concatenate([sa, sb], axis=1)


def _dense(sums, hist, W0, b0, W1, b1, W2, b2):
    w0t = W0.T
    w1t = W1.T
    w2t = W2.reshape(2 * F, 1)
    full = lambda shape: pl.BlockSpec(shape, lambda i: (0, 0))
    nblk = NP // BLK
    return pl.pallas_call(
        _dense_kernel,
        grid=(nblk,),
        in_specs=[
            pl.BlockSpec((BLK, F), lambda i: (i, 0)),
            pl.BlockSpec((BLK, F), lambda i: (i + nblk, 0)),
            pl.BlockSpec((NS, BLK), lambda i: (0, i)),
            pl.BlockSpec((NS, BLK), lambda i: (1, i)),
            full((2 * F, F)),
            pl.BlockSpec((F,), lambda i: (0,)),
            full((2 * F, F)),
            pl.BlockSpec((F,), lambda i: (0,)),
            full((2 * F, 1)),
            pl.BlockSpec((1,), lambda i: (0,)),
        ],
        out_specs=[
            pl.BlockSpec((BLK, F), lambda i: (i, 0)),
            pl.BlockSpec((BLK, 2), lambda i: (i, 0)),
        ],
        out_shape=[
            jax.ShapeDtypeStruct((NP, F), jnp.float32),
            jax.ShapeDtypeStruct((NP, 2), jnp.float32),
        ],
    )(sums, sums, hist, hist, w0t, b0, w1t, b1, w2t, b2)


def _gather_kernel(h1_hbm, tt_hbm, inter_hbm, emb_hbm, log_hbm,
                   inter_v, tt_v, log_v, ebuf, sem0, sem1):
    c = lax.axis_index("core")
    s = lax.axis_index("subcore")
    w = s * NC + c

    pltpu.sync_copy(inter_hbm.at[w], inter_v)
    pltpu.sync_copy(tt_hbm, tt_v)

    def start(j, buf, sem):
        pltpu.async_copy(h1_hbm.at[inter_v.at[j]], ebuf.at[buf], sem)

    def wait(buf, sem):
        pltpu.make_async_copy(h1_hbm.at[pl.ds(0, C3)],
                              ebuf.at[buf], sem).wait()

    start(0, 0, sem0)
    start(1, 1, sem1)

    lane = lax.iota(jnp.int32, L)

    @pl.loop(0, EPW // L)
    def _(i):
        p = i * (2 * L) + lane * 2
        prow = p // C3
        pcol = p % C3
        vs = plsc.load_gather(inter_v, [prow, pcol])
        vd = plsc.load_gather(inter_v, [prow, pcol + 1])
        va = plsc.load_gather(tt_v, [vs * 2])
        vb = plsc.load_gather(tt_v, [vd * 2 + 1])
        log_v[pl.ds(i * L, L)] = va + vb

    pltpu.sync_copy(log_v, log_hbm.at[pl.ds(w * EPW, EPW)])

    @pl.loop(0, ICH // 2)
    def _(k):
        wait(0, sem0)
        pltpu.sync_copy(ebuf.at[0],
                        emb_hbm.at[pl.ds(w * 2 * EPW + (2 * k) * C3,
                                         C3)])

        @pl.when(k < ICH // 2 - 1)
        def _():
            start(2 * k + 2, 0, sem0)

        wait(1, sem1)
        pltpu.sync_copy(ebuf.at[1],
                        emb_hbm.at[pl.ds(w * 2 * EPW + (2 * k + 1) * C3,
                                         C3)])

        @pl.when(k < ICH // 2 - 1)
        def _():
            start(2 * k + 3, 1, sem1)


def _gather(h1, tt, inter):
    mesh = plsc.VectorSubcoreMesh(core_axis_name="core",
                                  subcore_axis_name="subcore")
    f = pl.kernel(
        _gather_kernel,
        out_type=(jax.ShapeDtypeStruct((2 * E, F), jnp.float32),
                  jax.ShapeDtypeStruct((E,), jnp.float32)),
        mesh=mesh,
        scratch_types=[
            pltpu.VMEM((ICH, C3), jnp.int32),
            pltpu.VMEM((2 * NP,), jnp.float32),
            pltpu.VMEM((EPW,), jnp.float32),
            pltpu.VMEM((2, C3, F), jnp.float32),
            pltpu.SemaphoreType.DMA,
            pltpu.SemaphoreType.DMA,
        ],
        compiler_params=_SC_PARAMS,
    )
    return f(h1, tt, inter)


def kernel(edge_attr, edge_index, W0, b0, W1, b1, W2, b2):
    idx_chunks = edge_index.reshape(NC * NS * NSLAB, QTR, C1)
    inter = edge_index.T.reshape(NW, ICH, C3)

    sums, hist = _seg_sums(edge_attr, idx_chunks)
    h1, tt = _dense(sums, hist, W0, b0, W1, b1, W2, b2)
    emb2, logits = _gather(h1, tt.reshape(2 * NP), inter)
    return logits, emb2.reshape(E, 2 * F)


# speedup vs baseline: 6.5523x; 1.6879x over previous
import jax
import jax.numpy as jnp
from jax import lax
from jax.experimental import pallas as pl
from jax.experimental.pallas import tpu as pltpu
from jax.experimental.pallas import tpu_sc as plsc

_SC_PARAMS = pltpu.CompilerParams(needs_layout_passes=False)

E = 320000
N = 10000
F = 128
NP = 10240

NC = 2
NS = 16
L = 16

C1 = 40
NCH = (E // NS) // C1
NSLAB = 4
QTR = NCH // NSLAB
RPS = NP // NS

C3 = 80
NW = NC * NS
EPW = E // NW
ICH = (2 * EPW) // C3


def _seg_sum_kernel(ea_hbm, idx_hbm, acc_hbm, hist_hbm,
                    acc, idx_v, data_v, hist_v, sem0, sem1):
    c = lax.axis_index("core")
    s = lax.axis_index("subcore")
    cw = c * NS + s
    lane = lax.iota(jnp.int32, L)
    ones = jnp.ones((L,), jnp.float32)

    @pl.loop(0, C1)
    def _(i):
        @pl.loop(0, F, step=L)
        def _(l):
            data_v[0, i, pl.ds(l, L)] = jnp.zeros((L,), jnp.float32)

    @pl.loop(0, NP, step=L)
    def _(i):
        hist_v[0, pl.ds(i, L)] = jnp.zeros((L,), jnp.float32)

    @pl.loop(0, RPS // C1)
    def _(k):
        pltpu.sync_copy(data_v.at[0],
                        acc.at[pl.ds(s * RPS + k * C1, C1)])
    plsc.subcore_barrier()

    def start(g, buf, sem):
        pltpu.async_copy(ea_hbm.at[pl.ds((s * NCH + g) * C1, C1)],
                         data_v.at[buf], sem)

    def wait(buf, sem):
        pltpu.make_async_copy(ea_hbm.at[pl.ds(0, C1)],
                              data_v.at[buf], sem).wait()

    def scatter(j, buf):
        pltpu.sync_copy(data_v.at[buf], acc.at[idx_v.at[j]], add=True)

    for h in range(NSLAB):
        pltpu.sync_copy(idx_hbm.at[cw * NSLAB + h], idx_v)
        base = h * QTR
        start(base + 0, 0, sem0)
        start(base + 1, 1, sem1)

        @pl.loop(0, (QTR + 1) // 2)
        def _(k):
            wait(0, sem0)
            scatter(2 * k, 0)

            @pl.when(2 * k + 2 < QTR)
            def _():
                start(base + 2 * k + 2, 0, sem0)

            @pl.when(2 * k + 1 < QTR)
            def _():
                wait(1, sem1)
                scatter(2 * k + 1, 1)

                @pl.when(2 * k + 3 < QTR)
                def _():
                    start(base + 2 * k + 3, 1, sem1)

        @pl.loop(0, (QTR * C1 + L - 1) // L)
        def _(i):
            p = i * L + lane
            msk = p < QTR * C1
            vidx = plsc.load_gather(idx_v, [p // C1, p % C1], mask=msk)
            plsc.addupdate_scatter(hist_v, [jnp.zeros((L,), jnp.int32), vidx],
                                   ones, mask=msk)

    plsc.subcore_barrier()

    @pl.loop(0, RPS // C1)
    def _(k):
        pltpu.sync_copy(acc.at[pl.ds(s * RPS + k * C1, C1)], data_v.at[0])
        pltpu.sync_copy(data_v.at[0], acc_hbm.at[cw * (RPS // C1) + k])
    pltpu.sync_copy(hist_v, hist_hbm.at[cw])


def _seg_sums(edge_attr, idx_chunks):
    mesh = plsc.VectorSubcoreMesh(core_axis_name="core",
                                  subcore_axis_name="subcore")
    nslab = NC * NS * (RPS // C1)
    f = pl.kernel(
        _seg_sum_kernel,
        out_type=(jax.ShapeDtypeStruct((nslab, C1, F), jnp.float32),
                  jax.ShapeDtypeStruct((NC * NS, 1, NP), jnp.float32)),
        mesh=mesh,
        scratch_types=[
            pltpu.VMEM_SHARED((NP, F), jnp.float32),
            pltpu.VMEM((QTR, C1), jnp.int32),
            pltpu.VMEM((2, C1, F), jnp.float32),
            pltpu.VMEM((1, NP), jnp.float32),
            pltpu.SemaphoreType.DMA,
            pltpu.SemaphoreType.DMA,
        ],
        compiler_params=_SC_PARAMS,
    )
    accs, hist = f(edge_attr, idx_chunks)
    return accs.reshape(2 * NP, F), hist.reshape(NC * NS, NP)


BLK = 640


def _dense_kernel(sum_src, sum_dst, hist_src, hist_dst,
                  w0t, b0, w1t, b1, w2t, b2, h1_out, tt_out):
    cnt_src = jnp.sum(hist_src[...], axis=0).reshape(BLK, 1)
    cnt_dst = jnp.sum(hist_dst[...], axis=0).reshape(BLK, 1)
    mean_src = sum_src[...] / jnp.maximum(cnt_src, 1.0)
    mean_dst = sum_dst[...] / jnp.maximum(cnt_dst, 1.0)
    w0 = w0t[...]
    z0 = (jnp.dot(mean_dst, w0[:F], preferred_element_type=jnp.float32)
          + jnp.dot(mean_src, w0[F:], preferred_element_type=jnp.float32)
          + b0[...])
    h0 = 1.0 / (1.0 + jnp.exp(-z0))
    w1 = w1t[...]
    z1 = (jnp.dot(h0, w1[:F], preferred_element_type=jnp.float32)
          + jnp.dot(mean_src, w1[F:], preferred_element_type=jnp.float32)
          + b1[...])
    h1 = 1.0 / (1.0 + jnp.exp(-z1))
    h1_out[...] = h1
    w2 = w2t[...]
    sa = jnp.dot(h1, w2[:F], preferred_element_type=jnp.float32) + b2[...]
    sb = jnp.dot(h1, w2[F:], preferred_element_type=jnp.float32)
    tt_out[...] = jnp.concatenate([sa, sb], axis=1)


def _dense(sums, hist, W0, b0, W1, b1, W2, b2):
    w0t = W0.T
    w1t = W1.T
    w2t = W2.reshape(2 * F, 1)
    full = lambda shape: pl.BlockSpec(shape, lambda i: (0, 0))
    nblk = NP // BLK
    return pl.pallas_call(
        _dense_kernel,
        grid=(nblk,),
        in_specs=[
            pl.BlockSpec((BLK, F), lambda i: (i, 0)),
            pl.BlockSpec((BLK, F), lambda i: (i + nblk, 0)),
            pl.BlockSpec((NS, BLK), lambda i: (0, i)),
            pl.BlockSpec((NS, BLK), lambda i: (1, i)),
            full((2 * F, F)),
            pl.BlockSpec((F,), lambda i: (0,)),
            full((2 * F, F)),
            pl.BlockSpec((F,), lambda i: (0,)),
            full((2 * F, 1)),
            pl.BlockSpec((1,), lambda i: (0,)),
        ],
        out_specs=[
            pl.BlockSpec((BLK, F), lambda i: (i, 0)),
            pl.BlockSpec((BLK, 2), lambda i: (i, 0)),
        ],
        out_shape=[
            jax.ShapeDtypeStruct((NP, F), jnp.float32),
            jax.ShapeDtypeStruct((NP, 2), jnp.float32),
        ],
    )(sums, sums, hist, hist, w0t, b0, w1t, b1, w2t, b2)


def _gather_kernel(h1_hbm, tt_hbm, src_hbm, dst_hbm, emb_hbm, log_hbm,
                   src_v, dst_v, tt_v, log_v, ebuf, sem0, sem1):
    c = lax.axis_index("core")
    s = lax.axis_index("subcore")
    w = s * NC + c
    nch = EPW // C3

    pltpu.sync_copy(src_hbm.at[w], src_v)
    pltpu.sync_copy(dst_hbm.at[w], dst_v)
    pltpu.sync_copy(tt_hbm, tt_v)

    def start(j, buf, sem):
        pltpu.async_copy(h1_hbm.at[src_v.at[j]], ebuf.at[buf].at[0], sem)
        pltpu.async_copy(h1_hbm.at[dst_v.at[j]], ebuf.at[buf].at[1], sem)

    def wait(buf, sem):
        pltpu.make_async_copy(h1_hbm.at[pl.ds(0, 2 * C3)],
                              ebuf.at[buf], sem).wait()

    def store(j, buf):
        base = w * EPW + j * C3
        pltpu.sync_copy(ebuf.at[buf].at[0],
                        emb_hbm.at[pl.ds(base, C3), pl.ds(0, F)])
        pltpu.sync_copy(ebuf.at[buf].at[1],
                        emb_hbm.at[pl.ds(base, C3), pl.ds(F, F)])

    start(0, 0, sem0)
    start(1, 1, sem1)

    @pl.loop(0, nch)
    def _(r):
        @pl.loop(0, C3, step=L)
        def _(off):
            vs = src_v[r, pl.ds(off, L)]
            vd = dst_v[r, pl.ds(off, L)]
            va = plsc.load_gather(tt_v, [vs * 2])
            vb = plsc.load_gather(tt_v, [vd * 2 + 1])
            log_v[pl.ds(r * C3 + off, L)] = va + vb

    pltpu.sync_copy(log_v, log_hbm.at[pl.ds(w * EPW, EPW)])

    @pl.loop(0, nch // 2)
    def _(k):
        wait(0, sem0)
        store(2 * k, 0)

        @pl.when(2 * k + 2 < nch)
        def _():
            start(2 * k + 2, 0, sem0)

        wait(1, sem1)
        store(2 * k + 1, 1)

        @pl.when(2 * k + 3 < nch)
        def _():
            start(2 * k + 3, 1, sem1)

    if nch % 2:
        wait(0, sem0)
        store(nch - 1, 0)


def _gather(h1, tt, src_chunks, dst_chunks):
    mesh = plsc.VectorSubcoreMesh(core_axis_name="core",
                                  subcore_axis_name="subcore")
    f = pl.kernel(
        _gather_kernel,
        out_type=(jax.ShapeDtypeStruct((E, 2 * F), jnp.float32),
                  jax.ShapeDtypeStruct((E,), jnp.float32)),
        mesh=mesh,
        scratch_types=[
            pltpu.VMEM((EPW // C3, C3), jnp.int32),
            pltpu.VMEM((EPW // C3, C3), jnp.int32),
            pltpu.VMEM((2 * NP,), jnp.float32),
            pltpu.VMEM((EPW,), jnp.float32),
            pltpu.VMEM((2, 2, C3, F), jnp.float32),
            pltpu.SemaphoreType.DMA,
            pltpu.SemaphoreType.DMA,
        ],
        compiler_params=_SC_PARAMS,
    )
    return f(h1, tt, src_chunks, dst_chunks)


def kernel(edge_attr, edge_index, W0, b0, W1, b1, W2, b2):
    idx_chunks = edge_index.reshape(NC * NS * NSLAB, QTR, C1)
    src_chunks = edge_index[0].reshape(NW, EPW // C3, C3)
    dst_chunks = edge_index[1].reshape(NW, EPW // C3, C3)

    sums, hist = _seg_sums(edge_attr, idx_chunks)
    h1, tt = _dense(sums, hist, W0, b0, W1, b1, W2, b2)
    emb, logits = _gather(h1, tt.reshape(2 * NP), src_chunks, dst_chunks)
    return logits, emb
